# Initial kernel scaffold; baseline (speedup 1.0000x reference)
import jax
import jax.numpy as jnp
from jax.experimental import pallas as pl


def kernel(x, edge_index, W1_l, b1_l, W1_r, W2_l, b2_l, W2_r, W_lin, b_lin):
    raise NotImplementedError("write your pallas kernel here")


# speedup vs baseline: 6.1695x; 6.1695x over previous
import jax
import jax.numpy as jnp
from jax import lax
from jax.experimental import pallas as pl
from jax.experimental.pallas import tpu as pltpu
from jax.experimental.pallas import tpu_sc as plsc

N = 10000
E = 320000
D = 128
NS = 16
EPW = E // NS
K = 100
NWIN = EPW // K
NPAD = 10240
RPS = NPAD // NS
CW = 32
ZR = 128

_mesh = plsc.VectorSubcoreMesh(core_axis_name="c", subcore_axis_name="s",
                               num_cores=1)


def _make_sc_aggregate(with_count: bool):
    if with_count:
        out_type = [jax.ShapeDtypeStruct((NPAD, D), jnp.bfloat16),
                    jax.ShapeDtypeStruct((NPAD, CW), jnp.bfloat16)]
    else:
        out_type = jax.ShapeDtypeStruct((NPAD, D), jnp.bfloat16)

    scratch = [
        pltpu.VMEM((NWIN, K), jnp.int32),
        pltpu.VMEM((NWIN, K), jnp.int32),
        pltpu.VMEM((K, D), jnp.bfloat16),
        pltpu.VMEM((K, CW), jnp.bfloat16),
        pltpu.VMEM((ZR, D), jnp.bfloat16),
        pltpu.VMEM((ZR, CW), jnp.bfloat16),
        pltpu.VMEM_SHARED((NPAD, D), jnp.bfloat16),
    ]
    if with_count:
        scratch.append(pltpu.VMEM_SHARED((NPAD, CW), jnp.bfloat16))

    def body(x_hbm, src_hbm, dst_hbm, out_hbm, *rest):
        if with_count:
            cnt_hbm = rest[0]
            (srcv, dstv, rows, ones, zbuf, zcnt, acc_sh, cnt_sh) = rest[1:]
        else:
            cnt_hbm = cnt_sh = None
            (srcv, dstv, rows, ones, zbuf, zcnt, acc_sh) = rest

        sid = lax.axis_index("s")

        pltpu.sync_copy(src_hbm.at[sid], srcv)
        pltpu.sync_copy(dst_hbm.at[sid], dstv)

        @pl.loop(0, ZR, step=2)
        def _(r):
            r2 = pl.multiple_of(r, 2)
            for c in range(D // 16):
                zbuf[pl.ds(r2, 2), pl.ds(c * 16, 16)] = jnp.zeros(
                    (2, 16), jnp.bfloat16)
            for c in range(CW // 16):
                zcnt[pl.ds(r2, 2), pl.ds(c * 16, 16)] = jnp.zeros(
                    (2, 16), jnp.bfloat16)

        if with_count:
            @pl.loop(0, K, step=2)
            def _(r):
                r2 = pl.multiple_of(r, 2)
                for c in range(CW // 16):
                    ones[pl.ds(r2, 2), pl.ds(c * 16, 16)] = jnp.ones(
                        (2, 16), jnp.bfloat16)

        for j in range(RPS // ZR):
            base = sid * RPS + j * ZR
            pltpu.sync_copy(zbuf, acc_sh.at[pl.ds(base, ZR)])
            if with_count:
                pltpu.sync_copy(zcnt, cnt_sh.at[pl.ds(base, ZR)])
        plsc.subcore_barrier()

        @pl.loop(0, NWIN)
        def _(w):
            pltpu.sync_copy(x_hbm.at[srcv.at[w]], rows)
            pltpu.sync_copy(rows, acc_sh.at[dstv.at[w]], add=True)
            if with_count:
                pltpu.sync_copy(ones, cnt_sh.at[dstv.at[w]], add=True)

        plsc.subcore_barrier()

        pltpu.sync_copy(acc_sh.at[pl.ds(sid * RPS, RPS)],
                        out_hbm.at[pl.ds(sid * RPS, RPS)])
        if with_count:
            pltpu.sync_copy(cnt_sh.at[pl.ds(sid * RPS, RPS)],
                            cnt_hbm.at[pl.ds(sid * RPS, RPS)])

    return pl.kernel(body, out_type=out_type, mesh=_mesh,
                     scratch_types=scratch,
                     compiler_params=pltpu.CompilerParams(
                         use_tc_tiling_on_sc=False))


_sc_aggregate_cnt = _make_sc_aggregate(True)
_sc_aggregate = _make_sc_aggregate(False)


_R = 2000


def _tc_layer1(p, c, x, wl, wr, b):
    def body(p_ref, c_ref, x_ref, wl_ref, wr_ref, b_ref, o_ref, obf_ref):
        s = p_ref[...].astype(jnp.float32)
        cnt = c_ref[:, 0:1].astype(jnp.float32)
        mean = s / jnp.maximum(cnt, 1.0)
        h = (jnp.dot(mean, wl_ref[...], preferred_element_type=jnp.float32)
             + jnp.dot(x_ref[...], wr_ref[...],
                       preferred_element_type=jnp.float32)
             + b_ref[...])
        h = jnp.maximum(h, 0.0)
        o_ref[...] = h
        obf_ref[...] = h.astype(jnp.bfloat16)

    grid = (N // _R,)
    return pl.pallas_call(
        body,
        grid=grid,
        in_specs=[
            pl.BlockSpec((_R, D), lambda i: (i, 0)),
            pl.BlockSpec((_R, CW), lambda i: (i, 0)),
            pl.BlockSpec((_R, D), lambda i: (i, 0)),
            pl.BlockSpec((D, D), lambda i: (0, 0)),
            pl.BlockSpec((D, D), lambda i: (0, 0)),
            pl.BlockSpec((1, D), lambda i: (0, 0)),
        ],
        out_specs=[pl.BlockSpec((_R, D), lambda i: (i, 0)),
                   pl.BlockSpec((_R, D), lambda i: (i, 0))],
        out_shape=[jax.ShapeDtypeStruct((N, D), jnp.float32),
                   jax.ShapeDtypeStruct((N, D), jnp.bfloat16)],
    )(p, c, x, wl, wr, b)


def _tc_layer2(p, c, h, wl, wr, b, wlin, blin):
    def body(p_ref, c_ref, h_ref, wl_ref, wr_ref, b_ref, wo_ref, bo_ref,
             o_ref):
        s = p_ref[...].astype(jnp.float32)
        cnt = c_ref[:, 0:1].astype(jnp.float32)
        mean = s / jnp.maximum(cnt, 1.0)
        h2 = (jnp.dot(mean, wl_ref[...], preferred_element_type=jnp.float32)
              + jnp.dot(h_ref[...], wr_ref[...],
                        preferred_element_type=jnp.float32)
              + b_ref[...])
        h2 = jnp.maximum(h2, 0.0)
        o_ref[...] = (jnp.dot(h2, wo_ref[...],
                              preferred_element_type=jnp.float32)
                      + bo_ref[...])

    grid = (N // _R,)
    return pl.pallas_call(
        body,
        grid=grid,
        in_specs=[
            pl.BlockSpec((_R, D), lambda i: (i, 0)),
            pl.BlockSpec((_R, CW), lambda i: (i, 0)),
            pl.BlockSpec((_R, D), lambda i: (i, 0)),
            pl.BlockSpec((D, D), lambda i: (0, 0)),
            pl.BlockSpec((D, D), lambda i: (0, 0)),
            pl.BlockSpec((1, D), lambda i: (0, 0)),
            pl.BlockSpec((D, D), lambda i: (0, 0)),
            pl.BlockSpec((1, D), lambda i: (0, 0)),
        ],
        out_specs=pl.BlockSpec((_R, D), lambda i: (i, 0)),
        out_shape=jax.ShapeDtypeStruct((N, D), jnp.float32),
    )(p, c, h, wl, wr, b, wlin, blin)


def kernel(x, edge_index, W1_l, b1_l, W1_r, W2_l, b2_l, W2_r, W_lin, b_lin):
    src = edge_index[0].reshape(NS, NWIN, K)
    dst = edge_index[1].reshape(NS, NWIN, K)
    x_bf = x.astype(jnp.bfloat16)

    p1, c1 = _sc_aggregate_cnt(x_bf, src, dst)
    h1, h1_bf = _tc_layer1(p1[:N], c1[:N], x, W1_l.T, W1_r.T,
                           b1_l.reshape(1, D))
    p2 = _sc_aggregate(h1_bf, src, dst)
    out = _tc_layer2(p2[:N], c1[:N], h1, W2_l.T, W2_r.T, b2_l.reshape(1, D),
                     W_lin.T, b_lin.reshape(1, D))
    return out


# speedup vs baseline: 9.1442x; 1.4822x over previous
import jax
import jax.numpy as jnp
from jax import lax
from jax.experimental import pallas as pl
from jax.experimental.pallas import tpu as pltpu
from jax.experimental.pallas import tpu_sc as plsc

N = 10000
E = 320000
D = 128
NC = 2
NS = 16
NW = NC * NS
EPW = E // NW
K = 100
NWIN = EPW // K
NPAD = 10240
RPS = NPAD // NS
CW = 32
ZR = 128

_mesh = plsc.VectorSubcoreMesh(core_axis_name="c", subcore_axis_name="s",
                               num_cores=NC)


def _make_sc_aggregate(with_count: bool):
    if with_count:
        out_type = [jax.ShapeDtypeStruct((NC, NPAD, D), jnp.bfloat16),
                    jax.ShapeDtypeStruct((NC, NPAD, CW), jnp.bfloat16)]
    else:
        out_type = jax.ShapeDtypeStruct((NC, NPAD, D), jnp.bfloat16)

    scratch = [
        pltpu.VMEM((NWIN, K), jnp.int32),
        pltpu.VMEM((NWIN, K), jnp.int32),
        pltpu.VMEM((K, D), jnp.bfloat16),
        pltpu.VMEM((K, CW), jnp.bfloat16),
        pltpu.VMEM((ZR, D), jnp.bfloat16),
        pltpu.VMEM((ZR, CW), jnp.bfloat16),
        pltpu.VMEM_SHARED((NPAD, D), jnp.bfloat16),
    ]
    if with_count:
        scratch.append(pltpu.VMEM_SHARED((NPAD, CW), jnp.bfloat16))

    def body(x_hbm, src_hbm, dst_hbm, out_hbm, *rest):
        if with_count:
            cnt_hbm = rest[0]
            (srcv, dstv, rows, ones, zbuf, zcnt, acc_sh, cnt_sh) = rest[1:]
        else:
            cnt_hbm = cnt_sh = None
            (srcv, dstv, rows, ones, zbuf, zcnt, acc_sh) = rest

        cid = lax.axis_index("c")
        sid = lax.axis_index("s")
        wid = cid * NS + sid

        pltpu.sync_copy(src_hbm.at[wid], srcv)
        pltpu.sync_copy(dst_hbm.at[wid], dstv)

        @pl.loop(0, ZR, step=2)
        def _(r):
            r2 = pl.multiple_of(r, 2)
            for c in range(D // 16):
                zbuf[pl.ds(r2, 2), pl.ds(c * 16, 16)] = jnp.zeros(
                    (2, 16), jnp.bfloat16)
            for c in range(CW // 16):
                zcnt[pl.ds(r2, 2), pl.ds(c * 16, 16)] = jnp.zeros(
                    (2, 16), jnp.bfloat16)

        if with_count:
            @pl.loop(0, K, step=2)
            def _(r):
                r2 = pl.multiple_of(r, 2)
                for c in range(CW // 16):
                    ones[pl.ds(r2, 2), pl.ds(c * 16, 16)] = jnp.ones(
                        (2, 16), jnp.bfloat16)

        for j in range(RPS // ZR):
            base = sid * RPS + j * ZR
            pltpu.sync_copy(zbuf, acc_sh.at[pl.ds(base, ZR)])
            if with_count:
                pltpu.sync_copy(zcnt, cnt_sh.at[pl.ds(base, ZR)])
        plsc.subcore_barrier()

        @pl.loop(0, NWIN)
        def _(w):
            pltpu.sync_copy(x_hbm.at[srcv.at[w]], rows)
            pltpu.sync_copy(rows, acc_sh.at[dstv.at[w]], add=True)
            if with_count:
                pltpu.sync_copy(ones, cnt_sh.at[dstv.at[w]], add=True)

        plsc.subcore_barrier()

        pltpu.sync_copy(acc_sh.at[pl.ds(sid * RPS, RPS)],
                        out_hbm.at[cid, pl.ds(sid * RPS, RPS)])
        if with_count:
            pltpu.sync_copy(cnt_sh.at[pl.ds(sid * RPS, RPS)],
                            cnt_hbm.at[cid, pl.ds(sid * RPS, RPS)])

    return pl.kernel(body, out_type=out_type, mesh=_mesh,
                     scratch_types=scratch,
                     compiler_params=pltpu.CompilerParams(
                         use_tc_tiling_on_sc=False))


_sc_aggregate_cnt = _make_sc_aggregate(True)
_sc_aggregate = _make_sc_aggregate(False)


_R = 2000


def _tc_layer1(p, c, x, wl, wr, b):
    def body(p_ref, c_ref, x_ref, wl_ref, wr_ref, b_ref, o_ref, obf_ref):
        s = p_ref[0].astype(jnp.float32) + p_ref[1].astype(jnp.float32)
        cnt = (c_ref[0, :, 0:1].astype(jnp.float32)
               + c_ref[1, :, 0:1].astype(jnp.float32))
        mean = s / jnp.maximum(cnt, 1.0)
        h = (jnp.dot(mean, wl_ref[...], preferred_element_type=jnp.float32)
             + jnp.dot(x_ref[...], wr_ref[...],
                       preferred_element_type=jnp.float32)
             + b_ref[...])
        h = jnp.maximum(h, 0.0)
        o_ref[...] = h
        obf_ref[...] = h.astype(jnp.bfloat16)

    grid = (N // _R,)
    return pl.pallas_call(
        body,
        grid=grid,
        in_specs=[
            pl.BlockSpec((NC, _R, D), lambda i: (0, i, 0)),
            pl.BlockSpec((NC, _R, CW), lambda i: (0, i, 0)),
            pl.BlockSpec((_R, D), lambda i: (i, 0)),
            pl.BlockSpec((D, D), lambda i: (0, 0)),
            pl.BlockSpec((D, D), lambda i: (0, 0)),
            pl.BlockSpec((1, D), lambda i: (0, 0)),
        ],
        out_specs=[pl.BlockSpec((_R, D), lambda i: (i, 0)),
                   pl.BlockSpec((_R, D), lambda i: (i, 0))],
        out_shape=[jax.ShapeDtypeStruct((N, D), jnp.float32),
                   jax.ShapeDtypeStruct((N, D), jnp.bfloat16)],
    )(p, c, x, wl, wr, b)


def _tc_layer2(p, c, h, wl, wr, b, wlin, blin):
    def body(p_ref, c_ref, h_ref, wl_ref, wr_ref, b_ref, wo_ref, bo_ref,
             o_ref):
        s = p_ref[0].astype(jnp.float32) + p_ref[1].astype(jnp.float32)
        cnt = (c_ref[0, :, 0:1].astype(jnp.float32)
               + c_ref[1, :, 0:1].astype(jnp.float32))
        mean = s / jnp.maximum(cnt, 1.0)
        h2 = (jnp.dot(mean, wl_ref[...], preferred_element_type=jnp.float32)
              + jnp.dot(h_ref[...], wr_ref[...],
                        preferred_element_type=jnp.float32)
              + b_ref[...])
        h2 = jnp.maximum(h2, 0.0)
        o_ref[...] = (jnp.dot(h2, wo_ref[...],
                              preferred_element_type=jnp.float32)
                      + bo_ref[...])

    grid = (N // _R,)
    return pl.pallas_call(
        body,
        grid=grid,
        in_specs=[
            pl.BlockSpec((NC, _R, D), lambda i: (0, i, 0)),
            pl.BlockSpec((NC, _R, CW), lambda i: (0, i, 0)),
            pl.BlockSpec((_R, D), lambda i: (i, 0)),
            pl.BlockSpec((D, D), lambda i: (0, 0)),
            pl.BlockSpec((D, D), lambda i: (0, 0)),
            pl.BlockSpec((1, D), lambda i: (0, 0)),
            pl.BlockSpec((D, D), lambda i: (0, 0)),
            pl.BlockSpec((1, D), lambda i: (0, 0)),
        ],
        out_specs=pl.BlockSpec((_R, D), lambda i: (i, 0)),
        out_shape=jax.ShapeDtypeStruct((N, D), jnp.float32),
    )(p, c, h, wl, wr, b, wlin, blin)


def kernel(x, edge_index, W1_l, b1_l, W1_r, W2_l, b2_l, W2_r, W_lin, b_lin):
    src = edge_index[0].reshape(NW, NWIN, K)
    dst = edge_index[1].reshape(NW, NWIN, K)
    x_bf = x.astype(jnp.bfloat16)

    p1, c1 = _sc_aggregate_cnt(x_bf, src, dst)
    h1, h1_bf = _tc_layer1(p1[:, :N], c1[:, :N], x, W1_l.T, W1_r.T,
                           b1_l.reshape(1, D))
    p2 = _sc_aggregate(h1_bf, src, dst)
    out = _tc_layer2(p2[:, :N], c1[:, :N], h1, W2_l.T, W2_r.T,
                     b2_l.reshape(1, D), W_lin.T, b_lin.reshape(1, D))
    return out


# speedup vs baseline: 14.0003x; 1.5311x over previous
import jax
import jax.numpy as jnp
from jax import lax
from jax.experimental import pallas as pl
from jax.experimental.pallas import tpu as pltpu
from jax.experimental.pallas import tpu_sc as plsc

N = 10000
E = 320000
D = 128
NC = 2
NS = 16
NW = NC * NS
EPW = E // NW
K = 100
NWIN = EPW // K
NPAD = 10240
RPS = NPAD // NS
CW = 32
ZR = 128
S = 5

_mesh = plsc.VectorSubcoreMesh(core_axis_name="c", subcore_axis_name="s",
                               num_cores=NC)


def _make_sc_aggregate(with_count: bool):
    if with_count:
        out_type = [jax.ShapeDtypeStruct((NC, NPAD, D), jnp.bfloat16),
                    jax.ShapeDtypeStruct((NC, NPAD, CW), jnp.bfloat16)]
    else:
        out_type = jax.ShapeDtypeStruct((NC, NPAD, D), jnp.bfloat16)

    scratch = [
        pltpu.VMEM((NWIN, K), jnp.int32),
        pltpu.VMEM((NWIN, K), jnp.int32),
        pltpu.VMEM((K, CW), jnp.bfloat16),
        pltpu.VMEM((ZR, D), jnp.bfloat16),
        pltpu.VMEM((ZR, CW), jnp.bfloat16),
        pltpu.VMEM_SHARED((NPAD, D), jnp.bfloat16),
    ]
    if with_count:
        scratch.append(pltpu.VMEM_SHARED((NPAD, CW), jnp.bfloat16))
    scratch += [pltpu.VMEM((K, D), jnp.bfloat16)] * S
    scratch += [pltpu.SemaphoreType.DMA] * (3 * S)

    def body(x_hbm, src_hbm, dst_hbm, out_hbm, *rest):
        if with_count:
            cnt_hbm = rest[0]
            (srcv, dstv, ones, zbuf, zcnt, acc_sh, cnt_sh) = rest[1:8]
            rest = rest[8:]
        else:
            cnt_hbm = cnt_sh = None
            (srcv, dstv, ones, zbuf, zcnt, acc_sh) = rest[:6]
            rest = rest[6:]
        rows = rest[:S]
        gsem = rest[S:2 * S]
        ssem = rest[2 * S:3 * S]
        csem = rest[3 * S:4 * S]

        cid = lax.axis_index("c")
        sid = lax.axis_index("s")
        wid = cid * NS + sid

        pltpu.sync_copy(src_hbm.at[wid], srcv)
        pltpu.sync_copy(dst_hbm.at[wid], dstv)

        @pl.loop(0, ZR, step=2)
        def _(r):
            r2 = pl.multiple_of(r, 2)
            for c in range(D // 16):
                zbuf[pl.ds(r2, 2), pl.ds(c * 16, 16)] = jnp.zeros(
                    (2, 16), jnp.bfloat16)
            for c in range(CW // 16):
                zcnt[pl.ds(r2, 2), pl.ds(c * 16, 16)] = jnp.zeros(
                    (2, 16), jnp.bfloat16)

        if with_count:
            @pl.loop(0, K, step=2)
            def _(r):
                r2 = pl.multiple_of(r, 2)
                for c in range(CW // 16):
                    ones[pl.ds(r2, 2), pl.ds(c * 16, 16)] = jnp.ones(
                        (2, 16), jnp.bfloat16)

        for j in range(RPS // ZR):
            base = sid * RPS + j * ZR
            pltpu.sync_copy(zbuf, acc_sh.at[pl.ds(base, ZR)])
            if with_count:
                pltpu.sync_copy(zcnt, cnt_sh.at[pl.ds(base, ZR)])
        plsc.subcore_barrier()

        def wait_gather(b):
            pltpu.make_async_copy(x_hbm.at[pl.ds(0, K)], rows[b],
                                  gsem[b]).wait()

        for b in range(S):
            pltpu.async_copy(x_hbm.at[srcv.at[b]], rows[b], gsem[b])

        @pl.loop(0, NWIN - S, step=S)
        def _(g):
            scat = []
            for b in range(S):
                w = g + b
                wait_gather(b)
                d = pltpu.async_copy(rows[b], acc_sh.at[dstv.at[w]],
                                     ssem[b], add=True)
                c = (pltpu.async_copy(ones, cnt_sh.at[dstv.at[w]],
                                      csem[b], add=True)
                     if with_count else None)
                scat.append((d, c))
            for b in range(S):
                d, c = scat[b]
                d.wait()
                if c is not None:
                    c.wait()
                pltpu.async_copy(x_hbm.at[srcv.at[g + b + S]], rows[b],
                                 gsem[b])

        scat = []
        for b in range(S):
            w = NWIN - S + b
            wait_gather(b)
            d = pltpu.async_copy(rows[b], acc_sh.at[dstv.at[w]],
                                 ssem[b], add=True)
            c = (pltpu.async_copy(ones, cnt_sh.at[dstv.at[w]],
                                  csem[b], add=True)
                 if with_count else None)
            scat.append((d, c))
        for d, c in scat:
            d.wait()
            if c is not None:
                c.wait()

        plsc.subcore_barrier()

        pltpu.sync_copy(acc_sh.at[pl.ds(sid * RPS, RPS)],
                        out_hbm.at[cid, pl.ds(sid * RPS, RPS)])
        if with_count:
            pltpu.sync_copy(cnt_sh.at[pl.ds(sid * RPS, RPS)],
                            cnt_hbm.at[cid, pl.ds(sid * RPS, RPS)])

    return pl.kernel(body, out_type=out_type, mesh=_mesh,
                     scratch_types=scratch,
                     compiler_params=pltpu.CompilerParams(
                         use_tc_tiling_on_sc=False))


_sc_aggregate_cnt = _make_sc_aggregate(True)
_sc_aggregate = _make_sc_aggregate(False)


_R = 2000


def _tc_layer1(p, c, x, wl, wr, b):
    def body(p_ref, c_ref, x_ref, wl_ref, wr_ref, b_ref, o_ref, obf_ref):
        s = p_ref[0].astype(jnp.float32) + p_ref[1].astype(jnp.float32)
        cnt = (c_ref[0, :, 0:1].astype(jnp.float32)
               + c_ref[1, :, 0:1].astype(jnp.float32))
        mean = s / jnp.maximum(cnt, 1.0)
        h = (jnp.dot(mean, wl_ref[...], preferred_element_type=jnp.float32)
             + jnp.dot(x_ref[...], wr_ref[...],
                       preferred_element_type=jnp.float32)
             + b_ref[...])
        h = jnp.maximum(h, 0.0)
        o_ref[...] = h
        obf_ref[...] = h.astype(jnp.bfloat16)

    grid = (N // _R,)
    return pl.pallas_call(
        body,
        grid=grid,
        in_specs=[
            pl.BlockSpec((NC, _R, D), lambda i: (0, i, 0)),
            pl.BlockSpec((NC, _R, CW), lambda i: (0, i, 0)),
            pl.BlockSpec((_R, D), lambda i: (i, 0)),
            pl.BlockSpec((D, D), lambda i: (0, 0)),
            pl.BlockSpec((D, D), lambda i: (0, 0)),
            pl.BlockSpec((1, D), lambda i: (0, 0)),
        ],
        out_specs=[pl.BlockSpec((_R, D), lambda i: (i, 0)),
                   pl.BlockSpec((_R, D), lambda i: (i, 0))],
        out_shape=[jax.ShapeDtypeStruct((N, D), jnp.float32),
                   jax.ShapeDtypeStruct((N, D), jnp.bfloat16)],
    )(p, c, x, wl, wr, b)


def _tc_layer2(p, c, h, wl, wr, b, wlin, blin):
    def body(p_ref, c_ref, h_ref, wl_ref, wr_ref, b_ref, wo_ref, bo_ref,
             o_ref):
        s = p_ref[0].astype(jnp.float32) + p_ref[1].astype(jnp.float32)
        cnt = (c_ref[0, :, 0:1].astype(jnp.float32)
               + c_ref[1, :, 0:1].astype(jnp.float32))
        mean = s / jnp.maximum(cnt, 1.0)
        h2 = (jnp.dot(mean, wl_ref[...], preferred_element_type=jnp.float32)
              + jnp.dot(h_ref[...], wr_ref[...],
                        preferred_element_type=jnp.float32)
              + b_ref[...])
        h2 = jnp.maximum(h2, 0.0)
        o_ref[...] = (jnp.dot(h2, wo_ref[...],
                              preferred_element_type=jnp.float32)
                      + bo_ref[...])

    grid = (N // _R,)
    return pl.pallas_call(
        body,
        grid=grid,
        in_specs=[
            pl.BlockSpec((NC, _R, D), lambda i: (0, i, 0)),
            pl.BlockSpec((NC, _R, CW), lambda i: (0, i, 0)),
            pl.BlockSpec((_R, D), lambda i: (i, 0)),
            pl.BlockSpec((D, D), lambda i: (0, 0)),
            pl.BlockSpec((D, D), lambda i: (0, 0)),
            pl.BlockSpec((1, D), lambda i: (0, 0)),
            pl.BlockSpec((D, D), lambda i: (0, 0)),
            pl.BlockSpec((1, D), lambda i: (0, 0)),
        ],
        out_specs=pl.BlockSpec((_R, D), lambda i: (i, 0)),
        out_shape=jax.ShapeDtypeStruct((N, D), jnp.float32),
    )(p, c, h, wl, wr, b, wlin, blin)


def kernel(x, edge_index, W1_l, b1_l, W1_r, W2_l, b2_l, W2_r, W_lin, b_lin):
    src = edge_index[0].reshape(NW, NWIN, K)
    dst = edge_index[1].reshape(NW, NWIN, K)
    x_bf = x.astype(jnp.bfloat16)

    p1, c1 = _sc_aggregate_cnt(x_bf, src, dst)
    h1, h1_bf = _tc_layer1(p1[:, :N], c1[:, :N], x, W1_l.T, W1_r.T,
                           b1_l.reshape(1, D))
    p2 = _sc_aggregate(h1_bf, src, dst)
    out = _tc_layer2(p2[:, :N], c1[:, :N], h1, W2_l.T, W2_r.T,
                     b2_l.reshape(1, D), W_lin.T, b_lin.reshape(1, D))
    return out


# speedup vs baseline: 16.0088x; 1.1435x over previous
import jax
import jax.numpy as jnp
from jax import lax
from jax.experimental import pallas as pl
from jax.experimental.pallas import tpu as pltpu
from jax.experimental.pallas import tpu_sc as plsc

N = 10000
E = 320000
D = 128
NC = 2
NS = 16
NW = NC * NS
EPW = E // NW
K = 80
NWIN = EPW // K
NPAD = 10240
RPS = NPAD // NS
CW = 32
ZR = 128
S = 5

_mesh = plsc.VectorSubcoreMesh(core_axis_name="c", subcore_axis_name="s",
                               num_cores=NC)


def _make_sc_aggregate(with_count: bool):
    if with_count:
        out_type = [jax.ShapeDtypeStruct((NC, NPAD, D), jnp.bfloat16),
                    jax.ShapeDtypeStruct((NC, NPAD, CW), jnp.bfloat16)]
    else:
        out_type = jax.ShapeDtypeStruct((NC, NPAD, D), jnp.bfloat16)

    scratch = [
        pltpu.VMEM((EPW,), jnp.int32),
        pltpu.VMEM((EPW,), jnp.int32),
        pltpu.VMEM((K, CW), jnp.bfloat16),
        pltpu.VMEM((ZR, D), jnp.bfloat16),
        pltpu.VMEM((ZR, CW), jnp.bfloat16),
        pltpu.VMEM_SHARED((NPAD, D), jnp.bfloat16),
    ]
    if with_count:
        scratch.append(pltpu.VMEM_SHARED((NPAD, CW), jnp.bfloat16))
    scratch += [pltpu.VMEM((K, D), jnp.bfloat16)] * S
    scratch += [pltpu.SemaphoreType.DMA] * (3 * S)

    def body(x_hbm, eix_hbm, out_hbm, *rest):
        if with_count:
            cnt_hbm = rest[0]
            (srcv, dstv, ones, zbuf, zcnt, acc_sh, cnt_sh) = rest[1:8]
            rest = rest[8:]
        else:
            cnt_hbm = cnt_sh = None
            (srcv, dstv, ones, zbuf, zcnt, acc_sh) = rest[:6]
            rest = rest[6:]
        rows = rest[:S]
        gsem = rest[S:2 * S]
        ssem = rest[2 * S:3 * S]
        csem = rest[3 * S:4 * S]

        cid = lax.axis_index("c")
        sid = lax.axis_index("s")
        wid = cid * NS + sid

        pltpu.sync_copy(eix_hbm.at[0, pl.ds(wid * EPW, EPW)], srcv)
        pltpu.sync_copy(eix_hbm.at[1, pl.ds(wid * EPW, EPW)], dstv)

        @pl.loop(0, ZR, step=2)
        def _(r):
            r2 = pl.multiple_of(r, 2)
            for c in range(D // 16):
                zbuf[pl.ds(r2, 2), pl.ds(c * 16, 16)] = jnp.zeros(
                    (2, 16), jnp.bfloat16)
            for c in range(CW // 16):
                zcnt[pl.ds(r2, 2), pl.ds(c * 16, 16)] = jnp.zeros(
                    (2, 16), jnp.bfloat16)

        if with_count:
            @pl.loop(0, K, step=2)
            def _(r):
                r2 = pl.multiple_of(r, 2)
                for c in range(CW // 16):
                    ones[pl.ds(r2, 2), pl.ds(c * 16, 16)] = jnp.ones(
                        (2, 16), jnp.bfloat16)

        for j in range(RPS // ZR):
            base = sid * RPS + j * ZR
            pltpu.sync_copy(zbuf, acc_sh.at[pl.ds(base, ZR)])
            if with_count:
                pltpu.sync_copy(zcnt, cnt_sh.at[pl.ds(base, ZR)])
        plsc.subcore_barrier()

        def wait_gather(b):
            pltpu.make_async_copy(x_hbm.at[pl.ds(0, K)], rows[b],
                                  gsem[b]).wait()

        def sidx(w):
            return srcv.at[pl.ds(w * K, K)]

        def didx(w):
            return dstv.at[pl.ds(w * K, K)]

        for b in range(S):
            pltpu.async_copy(x_hbm.at[sidx(b)], rows[b], gsem[b])

        @pl.loop(0, NWIN - S, step=S)
        def _(g):
            scat = []
            for b in range(S):
                w = g + b
                wait_gather(b)
                d = pltpu.async_copy(rows[b], acc_sh.at[didx(w)],
                                     ssem[b], add=True)
                c = (pltpu.async_copy(ones, cnt_sh.at[didx(w)],
                                      csem[b], add=True)
                     if with_count else None)
                scat.append((d, c))
            for b in range(S):
                d, c = scat[b]
                d.wait()
                if c is not None:
                    c.wait()
                pltpu.async_copy(x_hbm.at[sidx(g + b + S)], rows[b],
                                 gsem[b])

        scat = []
        for b in range(S):
            w = NWIN - S + b
            wait_gather(b)
            d = pltpu.async_copy(rows[b], acc_sh.at[didx(w)],
                                 ssem[b], add=True)
            c = (pltpu.async_copy(ones, cnt_sh.at[didx(w)],
                                  csem[b], add=True)
                 if with_count else None)
            scat.append((d, c))
        for d, c in scat:
            d.wait()
            if c is not None:
                c.wait()

        plsc.subcore_barrier()

        pltpu.sync_copy(acc_sh.at[pl.ds(sid * RPS, RPS)],
                        out_hbm.at[cid, pl.ds(sid * RPS, RPS)])
        if with_count:
            pltpu.sync_copy(cnt_sh.at[pl.ds(sid * RPS, RPS)],
                            cnt_hbm.at[cid, pl.ds(sid * RPS, RPS)])

    return pl.kernel(body, out_type=out_type, mesh=_mesh,
                     scratch_types=scratch,
                     compiler_params=pltpu.CompilerParams(
                         use_tc_tiling_on_sc=False))


_sc_aggregate_cnt = _make_sc_aggregate(True)
_sc_aggregate = _make_sc_aggregate(False)


_R = 2000


def _tc_layer1(p, c, x, wl, wr, b):
    def body(p_ref, c_ref, x_ref, wl_ref, wr_ref, b_ref, o_ref, obf_ref):
        s = p_ref[0].astype(jnp.float32) + p_ref[1].astype(jnp.float32)
        cnt = (c_ref[0, :, 0:1].astype(jnp.float32)
               + c_ref[1, :, 0:1].astype(jnp.float32))
        mean = s / jnp.maximum(cnt, 1.0)
        h = (jnp.dot(mean, wl_ref[...], preferred_element_type=jnp.float32)
             + jnp.dot(x_ref[...], wr_ref[...],
                       preferred_element_type=jnp.float32)
             + b_ref[...])
        h = jnp.maximum(h, 0.0)
        o_ref[...] = h
        obf_ref[...] = h.astype(jnp.bfloat16)

    grid = (N // _R,)
    return pl.pallas_call(
        body,
        grid=grid,
        in_specs=[
            pl.BlockSpec((NC, _R, D), lambda i: (0, i, 0)),
            pl.BlockSpec((NC, _R, CW), lambda i: (0, i, 0)),
            pl.BlockSpec((_R, D), lambda i: (i, 0)),
            pl.BlockSpec((D, D), lambda i: (0, 0)),
            pl.BlockSpec((D, D), lambda i: (0, 0)),
            pl.BlockSpec((1, D), lambda i: (0, 0)),
        ],
        out_specs=[pl.BlockSpec((_R, D), lambda i: (i, 0)),
                   pl.BlockSpec((_R, D), lambda i: (i, 0))],
        out_shape=[jax.ShapeDtypeStruct((N, D), jnp.float32),
                   jax.ShapeDtypeStruct((N, D), jnp.bfloat16)],
    )(p, c, x, wl, wr, b)


def _tc_layer2(p, c, h, wl, wr, b, wlin, blin):
    def body(p_ref, c_ref, h_ref, wl_ref, wr_ref, b_ref, wo_ref, bo_ref,
             o_ref):
        s = p_ref[0].astype(jnp.float32) + p_ref[1].astype(jnp.float32)
        cnt = (c_ref[0, :, 0:1].astype(jnp.float32)
               + c_ref[1, :, 0:1].astype(jnp.float32))
        mean = s / jnp.maximum(cnt, 1.0)
        h2 = (jnp.dot(mean, wl_ref[...], preferred_element_type=jnp.float32)
              + jnp.dot(h_ref[...], wr_ref[...],
                        preferred_element_type=jnp.float32)
              + b_ref[...])
        h2 = jnp.maximum(h2, 0.0)
        o_ref[...] = (jnp.dot(h2, wo_ref[...],
                              preferred_element_type=jnp.float32)
                      + bo_ref[...])

    grid = (N // _R,)
    return pl.pallas_call(
        body,
        grid=grid,
        in_specs=[
            pl.BlockSpec((NC, _R, D), lambda i: (0, i, 0)),
            pl.BlockSpec((NC, _R, CW), lambda i: (0, i, 0)),
            pl.BlockSpec((_R, D), lambda i: (i, 0)),
            pl.BlockSpec((D, D), lambda i: (0, 0)),
            pl.BlockSpec((D, D), lambda i: (0, 0)),
            pl.BlockSpec((1, D), lambda i: (0, 0)),
            pl.BlockSpec((D, D), lambda i: (0, 0)),
            pl.BlockSpec((1, D), lambda i: (0, 0)),
        ],
        out_specs=pl.BlockSpec((_R, D), lambda i: (i, 0)),
        out_shape=jax.ShapeDtypeStruct((N, D), jnp.float32),
    )(p, c, h, wl, wr, b, wlin, blin)


def kernel(x, edge_index, W1_l, b1_l, W1_r, W2_l, b2_l, W2_r, W_lin, b_lin):
    x_bf = x.astype(jnp.bfloat16)

    p1, c1 = _sc_aggregate_cnt(x_bf, edge_index)
    h1, h1_bf = _tc_layer1(p1, c1, x, W1_l.T, W1_r.T, b1_l.reshape(1, D))
    p2 = _sc_aggregate(h1_bf, edge_index)
    out = _tc_layer2(p2, c1, h1, W2_l.T, W2_r.T, b2_l.reshape(1, D),
                     W_lin.T, b_lin.reshape(1, D))
    return out
